# baseline (speedup 1.0000x reference)
import dataclasses
import functools

import jax
import jax.numpy as jnp
from jax import lax
from jax.experimental import pallas as pl
from jax.experimental.pallas import tpu as pltpu
from jax.experimental.pallas import tpu_sc as plsc

_NUM_CORES = 2
_NUM_SUBCORES = 16
_LANES = 16
_NW = _NUM_CORES * _NUM_SUBCORES

_W_BI = 0.3
_W_TRI = 0.15
_W_FOUR = 0.1

_K = 2
_UNROLL = 8
_SC_TOKENS = 2048
_TB = 8


def _bucket_fold(x, buckets):
    if buckets & (buckets - 1) == 0:
        return x & (buckets - 1)
    return x % buckets


def _sc_part(prev1, prev2, prev3, bi, tri, four, s_tok, v):
    tri_buckets = tri.shape[0]
    four_buckets = four.shape[0]
    b_per_w = s_tok // _NW
    n_chunks = b_per_w // _K
    assert b_per_w % (_LANES * 2 * _K) == 0

    mesh = plsc.VectorSubcoreMesh(core_axis_name="c", subcore_axis_name="s")

    compiler_params = pltpu.CompilerParams()
    if "needs_layout_passes" in pltpu.CompilerParams.__dataclass_fields__:
        compiler_params = dataclasses.replace(
            compiler_params, needs_layout_passes=False)

    @functools.partial(
        pl.kernel,
        out_type=jax.ShapeDtypeStruct((s_tok, v), jnp.float32),
        mesh=mesh,
        compiler_params=compiler_params,
        scratch_types=[
            pltpu.VMEM((b_per_w,), jnp.int32),
            pltpu.VMEM((b_per_w,), jnp.int32),
            pltpu.VMEM((b_per_w,), jnp.int32),
            pltpu.VMEM((n_chunks * 8,), jnp.int32),
            pltpu.VMEM((n_chunks * 8,), jnp.int32),
            pltpu.VMEM((n_chunks * 8,), jnp.int32),
            pltpu.VMEM((2, _K, v), jnp.float32),
            pltpu.VMEM((2, _K, v), jnp.float32),
            pltpu.VMEM((2, _K, v), jnp.float32),
            pltpu.SemaphoreType.DMA,
            pltpu.SemaphoreType.DMA,
            pltpu.SemaphoreType.DMA,
            pltpu.SemaphoreType.DMA,
        ],
    )
    def sc_kernel(p1_hbm, p2_hbm, p3_hbm, bi_hbm, tri_hbm, four_hbm, out_hbm,
                  in1, in2, in3, idx1, idx3, idx4, buf_b, buf_t, buf_f,
                  gsem0, gsem1, osem0, osem1):
        wid = lax.axis_index("s") * _NUM_CORES + lax.axis_index("c")
        base = wid * b_per_w
        gsems = (gsem0, gsem1)
        osems = (osem0, osem1)

        pltpu.sync_copy(p1_hbm.at[pl.ds(base, b_per_w)], in1)
        pltpu.sync_copy(p2_hbm.at[pl.ds(base, b_per_w)], in2)
        pltpu.sync_copy(p3_hbm.at[pl.ds(base, b_per_w)], in3)

        lane = lax.iota(jnp.int32, _LANES)
        slot = (lane // _K) * 8 + lax.rem(lane, _K)

        @pl.loop(0, b_per_w, step=_LANES)
        def _(i):
            s = pl.ds(i, _LANES)
            p1 = in1[s]
            p2 = in2[s]
            p3 = in3[s]
            partial_hash = 36313 * p1 + 27191 * p2
            h3 = _bucket_fold(partial_hash, tri_buckets)
            h4 = _bucket_fold(partial_hash + 51497 * p3, four_buckets)
            pos = slot + (i // _K) * 8
            plsc.store_scatter(idx1, [pos], p1)
            plsc.store_scatter(idx3, [pos], h3)
            plsc.store_scatter(idx4, [pos], h4)

        def gather_copies(g, si):
            go = g * 8
            return (
                pltpu.make_async_copy(
                    bi_hbm.at[idx1.at[pl.ds(go, _K)]], buf_b.at[si], gsems[si]),
                pltpu.make_async_copy(
                    tri_hbm.at[idx3.at[pl.ds(go, _K)]], buf_t.at[si], gsems[si]),
                pltpu.make_async_copy(
                    four_hbm.at[idx4.at[pl.ds(go, _K)]], buf_f.at[si], gsems[si]),
            )

        def out_copy(g, si):
            return pltpu.make_async_copy(
                buf_b.at[si], out_hbm.at[pl.ds(base + g * _K, _K)], osems[si])

        def issue_gathers(g, si):
            for c in gather_copies(g, si):
                c.start()

        def wait_gathers(g, si):
            for c in gather_copies(g, si):
                c.wait()

        def compute(si):
            for r in range(_K):
                @plsc.parallel_loop(0, v, step=_LANES, unroll=_UNROLL)
                def _(c):
                    cs = pl.ds(c, _LANES)
                    buf_b[si, r, cs] = (_W_BI * buf_b[si, r, cs]
                                        + _W_TRI * buf_t[si, r, cs]
                                        + _W_FOUR * buf_f[si, r, cs])

        issue_gathers(0, 0)

        @pl.loop(0, n_chunks, step=2)
        def _(g):
            issue_gathers(g + 1, 1)
            wait_gathers(g, 0)

            @pl.when(g >= 2)
            def _():
                out_copy(g - 2, 0).wait()

            compute(0)
            out_copy(g, 0).start()

            @pl.when(g + 2 < n_chunks)
            def _():
                issue_gathers(g + 2, 0)

            wait_gathers(g + 1, 1)

            @pl.when(g >= 2)
            def _():
                out_copy(g - 1, 1).wait()

            compute(1)
            out_copy(g + 1, 1).start()

        out_copy(n_chunks - 2, 0).wait()
        out_copy(n_chunks - 1, 1).wait()

    return sc_kernel(prev1, prev2, prev3, bi, tri, four)


def _tc_part(idx1, idx3, idx4, bi, tri, four, n, v, s0):
    g_steps = (n - s0) // _TB
    assert g_steps * _TB == n - s0 and s0 % _TB == 0

    def body(i1, i3, i4, *refs):
        del i1, i3, i4
        ins = refs[:-1]
        out_ref = refs[-1]
        for u in range(_TB):
            out_ref[u, :] = (_W_BI * ins[u][0, 0, :]
                            + _W_TRI * ins[_TB + u][0, 0, :]
                            + _W_FOUR * ins[2 * _TB + u][0, 0, :])

    in_specs = []
    for tab in range(3):
        for u in range(_TB):
            def imap(i, i1, i3, i4, u=u, tab=tab):
                sref = (i1, i3, i4)[tab]
                return (sref[i * _TB + u], 0, 0)
            in_specs.append(pl.BlockSpec((1, 1, v), imap))

    grid_spec = pltpu.PrefetchScalarGridSpec(
        num_scalar_prefetch=3,
        grid=(g_steps,),
        in_specs=in_specs,
        out_specs=pl.BlockSpec((_TB, v), lambda i, *_: (i + s0 // _TB, 0)),
    )
    bi3 = bi.reshape(bi.shape[0], 1, v)
    tri3 = tri.reshape(tri.shape[0], 1, v)
    four3 = four.reshape(four.shape[0], 1, v)
    return pl.pallas_call(
        body,
        grid_spec=grid_spec,
        out_shape=jax.ShapeDtypeStruct((n, v), jnp.float32),
    )(idx1, idx3, idx4, *([bi3] * _TB + [tri3] * _TB + [four3] * _TB))


def kernel(input_ids, bigram_table, trigram_table, fourgram_table):
    b, t = input_ids.shape
    n = b * t
    v = bigram_table.shape[1]
    assert 0 < _SC_TOKENS <= n

    flat = input_ids.reshape(-1).astype(jnp.int32)
    zero1 = jnp.zeros((1,), jnp.int32)
    prev2 = jnp.concatenate([zero1, flat[:-1]])
    prev3 = jnp.concatenate([zero1, zero1, flat[:-2]])

    sc_out = _sc_part(flat, prev2, prev3, bigram_table, trigram_table,
                      fourgram_table, _SC_TOKENS, v)
    if _SC_TOKENS == n:
        return sc_out

    s0 = _SC_TOKENS
    p1_t = flat[s0:]
    p2_t = prev2[s0:]
    p3_t = prev3[s0:]
    partial_hash = 36313 * p1_t + 27191 * p2_t
    h3_t = _bucket_fold(partial_hash, trigram_table.shape[0])
    h4_t = _bucket_fold(partial_hash + 51497 * p3_t, fourgram_table.shape[0])

    tc_full = _tc_part(p1_t, h3_t, h4_t, bigram_table, trigram_table,
                       fourgram_table, n, v, s0)
    return lax.dynamic_update_slice(tc_full, sc_out, (0, 0))


# speedup vs baseline: 8.3788x; 8.3788x over previous
import dataclasses
import functools

import jax
import jax.numpy as jnp
from jax import lax
from jax.experimental import pallas as pl
from jax.experimental.pallas import tpu as pltpu
from jax.experimental.pallas import tpu_sc as plsc

_NUM_CORES = 2
_NUM_SUBCORES = 16
_LANES = 16
_NW = _NUM_CORES * _NUM_SUBCORES

_W_BI = 0.3
_W_TRI = 0.15
_W_FOUR = 0.1

_K = 2
_UNROLL = 8


def _bucket_fold(x, buckets):
    if buckets & (buckets - 1) == 0:
        return x & (buckets - 1)
    return x % buckets


def kernel(input_ids, bigram_table, trigram_table, fourgram_table):
    b, t = input_ids.shape
    n = b * t
    v = bigram_table.shape[1]
    tri_buckets = trigram_table.shape[0]
    four_buckets = fourgram_table.shape[0]
    assert n % (_NW * _K * 2) == 0 and v % (_LANES * _UNROLL) == 0
    b_per_w = n // _NW
    n_chunks = b_per_w // _K

    flat = input_ids.reshape(-1).astype(jnp.int32)
    zero1 = jnp.zeros((1,), jnp.int32)
    prev2 = jnp.concatenate([zero1, flat[:-1]])
    prev3 = jnp.concatenate([zero1, zero1, flat[:-2]])

    mesh = plsc.VectorSubcoreMesh(core_axis_name="c", subcore_axis_name="s")

    compiler_params = pltpu.CompilerParams()
    if "needs_layout_passes" in pltpu.CompilerParams.__dataclass_fields__:
        compiler_params = dataclasses.replace(
            compiler_params, needs_layout_passes=False)

    @functools.partial(
        pl.kernel,
        out_type=jax.ShapeDtypeStruct((n, v), jnp.float32),
        mesh=mesh,
        compiler_params=compiler_params,
        scratch_types=[
            pltpu.VMEM((b_per_w,), jnp.int32),
            pltpu.VMEM((b_per_w,), jnp.int32),
            pltpu.VMEM((b_per_w,), jnp.int32),
            pltpu.VMEM((n_chunks * 8,), jnp.int32),
            pltpu.VMEM((n_chunks * 8,), jnp.int32),
            pltpu.VMEM((n_chunks * 8,), jnp.int32),
            pltpu.VMEM((2, _K, v), jnp.float32),
            pltpu.VMEM((2, _K, v), jnp.float32),
            pltpu.VMEM((2, _K, v), jnp.float32),
            pltpu.SemaphoreType.DMA,
            pltpu.SemaphoreType.DMA,
            pltpu.SemaphoreType.DMA,
            pltpu.SemaphoreType.DMA,
        ],
    )
    def sc_kernel(p1_hbm, p2_hbm, p3_hbm, bi_hbm, tri_hbm, four_hbm, out_hbm,
                  in1, in2, in3, idx1, idx3, idx4, buf_b, buf_t, buf_f,
                  gsem0, gsem1, osem0, osem1):
        wid = lax.axis_index("s") * _NUM_CORES + lax.axis_index("c")
        base = wid * b_per_w
        gsems = (gsem0, gsem1)
        osems = (osem0, osem1)

        pltpu.sync_copy(p1_hbm.at[pl.ds(base, b_per_w)], in1)
        pltpu.sync_copy(p2_hbm.at[pl.ds(base, b_per_w)], in2)
        pltpu.sync_copy(p3_hbm.at[pl.ds(base, b_per_w)], in3)

        lane = lax.iota(jnp.int32, _LANES)
        slot = (lane // _K) * 8 + lax.rem(lane, _K)

        @pl.loop(0, b_per_w, step=_LANES)
        def _(i):
            s = pl.ds(i, _LANES)
            p1 = in1[s]
            p2 = in2[s]
            p3 = in3[s]
            partial_hash = 36313 * p1 + 27191 * p2
            h3 = _bucket_fold(partial_hash, tri_buckets)
            h4 = _bucket_fold(partial_hash + 51497 * p3, four_buckets)
            pos = slot + (i // _K) * 8
            plsc.store_scatter(idx1, [pos], p1)
            plsc.store_scatter(idx3, [pos], h3)
            plsc.store_scatter(idx4, [pos], h4)

        def gather_copies(g, si):
            go = g * 8
            return (
                pltpu.make_async_copy(
                    bi_hbm.at[idx1.at[pl.ds(go, _K)]], buf_b.at[si], gsems[si]),
                pltpu.make_async_copy(
                    tri_hbm.at[idx3.at[pl.ds(go, _K)]], buf_t.at[si], gsems[si]),
                pltpu.make_async_copy(
                    four_hbm.at[idx4.at[pl.ds(go, _K)]], buf_f.at[si], gsems[si]),
            )

        def out_copy(g, si):
            return pltpu.make_async_copy(
                buf_b.at[si], out_hbm.at[pl.ds(base + g * _K, _K)], osems[si])

        def issue_gathers(g, si):
            for c in gather_copies(g, si):
                c.start()

        def wait_gathers(g, si):
            for c in gather_copies(g, si):
                c.wait()

        def compute(si):
            for r in range(_K):
                @plsc.parallel_loop(0, v, step=_LANES, unroll=_UNROLL)
                def _(c):
                    cs = pl.ds(c, _LANES)
                    buf_b[si, r, cs] = (_W_BI * buf_b[si, r, cs]
                                        + _W_TRI * buf_t[si, r, cs]
                                        + _W_FOUR * buf_f[si, r, cs])

        issue_gathers(0, 0)

        @pl.loop(0, n_chunks, step=2)
        def _(g):
            issue_gathers(g + 1, 1)
            wait_gathers(g, 0)

            @pl.when(g >= 2)
            def _():
                out_copy(g - 2, 0).wait()

            compute(0)
            out_copy(g, 0).start()

            @pl.when(g + 2 < n_chunks)
            def _():
                issue_gathers(g + 2, 0)

            wait_gathers(g + 1, 1)

            @pl.when(g >= 2)
            def _():
                out_copy(g - 1, 1).wait()

            compute(1)
            out_copy(g + 1, 1).start()

        out_copy(n_chunks - 2, 0).wait()
        out_copy(n_chunks - 1, 1).wait()

    return sc_kernel(flat, prev2, prev3, bigram_table, trigram_table,
                     fourgram_table)
